# baseline (speedup 1.0000x reference)
import functools

import jax
import jax.numpy as jnp
from jax import lax
from jax.experimental import pallas as pl
from jax.experimental.pallas import tpu as pltpu
from jax.experimental.pallas import tpu_sc as plsc

VOCAB = 100000
DEPTH = 64
LINE = 2 * DEPTH
B_TOTAL = 4096 * 200
CHUNK = 128
N_CHUNKS = B_TOTAL // CHUNK
NC, NS = 2, 16
NW = NC * NS
CPW = N_CHUNKS // NW
NBUF = 5
GROUPS = CPW // NBUF


def _make_gather():
    mesh = plsc.VectorSubcoreMesh(core_axis_name="c", subcore_axis_name="s")

    @functools.partial(
        pl.kernel,
        mesh=mesh,
        out_type=jax.ShapeDtypeStruct((B_TOTAL, LINE), jnp.float32),
        scratch_types=[
            pltpu.VMEM((CPW, CHUNK), jnp.float32),
            [pltpu.VMEM((CHUNK, LINE), jnp.float32) for _ in range(NBUF)],
            [pltpu.SemaphoreType.DMA for _ in range(NBUF)],
            [pltpu.SemaphoreType.DMA for _ in range(NBUF)],
        ],
        compiler_params=pltpu.CompilerParams(use_tc_tiling_on_sc=True),
    )
    def gather_kernel(table_hbm, idx_hbm, out_hbm, idx_f, rows, gsem, ssem):
        wid = lax.axis_index("s") * NC + lax.axis_index("c")
        base = wid * CPW
        pltpu.sync_copy(idx_hbm.at[pl.ds(base, CPW)], idx_f)
        idx_v = idx_f.bitcast(jnp.int32)

        def gather(j, b):
            pltpu.make_async_copy(
                table_hbm.at[idx_v.at[j]], rows[b], gsem[b]
            ).start()

        def gather_wait(b):
            pltpu.make_async_copy(
                table_hbm.at[idx_v.at[0]], rows[b], gsem[b]
            ).wait()

        def store(j, b):
            pltpu.make_async_copy(
                rows[b], out_hbm.at[pl.ds((base + j) * CHUNK, CHUNK)], ssem[b]
            ).start()

        def store_wait(b):
            pltpu.make_async_copy(
                rows[b], out_hbm.at[pl.ds(0, CHUNK)], ssem[b]
            ).wait()

        for b in range(NBUF):
            gather(b, b)

        def group(g, carry):
            for b in range(NBUF):
                gather_wait(b)
                store(g * NBUF + b, b)
            for b in range(NBUF):
                store_wait(b)
                gather((g + 1) * NBUF + b, b)
            return carry

        lax.fori_loop(0, GROUPS - 1, group, 0)

        for b in range(NBUF):
            gather_wait(b)
            store((GROUPS - 1) * NBUF + b, b)
        for b in range(NBUF):
            store_wait(b)

    return gather_kernel


_gather = _make_gather()


@jax.jit
def kernel(x, embeddings):
    t4 = jnp.concatenate([embeddings, jnp.roll(embeddings, -1, axis=0)], axis=1)
    idx = jax.lax.bitcast_convert_type(x.reshape(N_CHUNKS, CHUNK), jnp.float32)
    lines = _gather(t4, idx)
    out = lines.reshape(x.shape[0], x.shape[1], 2 * DEPTH)
    return jax.lax.slice_in_dim(out, 0, DEPTH, axis=2)


# speedup vs baseline: 1.0040x; 1.0040x over previous
import functools

import jax
import jax.numpy as jnp
from jax import lax
from jax.experimental import pallas as pl
from jax.experimental.pallas import tpu as pltpu
from jax.experimental.pallas import tpu_sc as plsc

VOCAB = 100000
DEPTH = 64
LINE = 2 * DEPTH
B_TOTAL = 4096 * 200
CHUNK = 128
N_CHUNKS = B_TOTAL // CHUNK
NC, NS = 2, 16
NW = NC * NS
CPW = N_CHUNKS // NW
NBUF = 4
GROUPS = CPW // NBUF


def _make_gather():
    mesh = plsc.VectorSubcoreMesh(core_axis_name="c", subcore_axis_name="s")

    @functools.partial(
        pl.kernel,
        mesh=mesh,
        out_type=jax.ShapeDtypeStruct((B_TOTAL, LINE), jnp.float32),
        scratch_types=[
            pltpu.VMEM((CPW, CHUNK), jnp.int32),
            [pltpu.VMEM((CHUNK, LINE), jnp.float32) for _ in range(NBUF)],
            [pltpu.SemaphoreType.DMA for _ in range(NBUF)],
            [pltpu.SemaphoreType.DMA for _ in range(NBUF)],
        ],
        compiler_params=pltpu.CompilerParams(use_tc_tiling_on_sc=True),
    )
    def gather_kernel(table_hbm, idx_hbm, out_hbm, idx_v, rows, gsem, ssem):
        wid = lax.axis_index("s") * NC + lax.axis_index("c")
        base = wid * CPW
        pltpu.sync_copy(idx_hbm.at[pl.ds(base, CPW)], idx_v)

        def gather(j, b):
            pltpu.make_async_copy(
                table_hbm.at[idx_v.at[j]], rows[b], gsem[b]
            ).start()

        def gather_wait(b):
            pltpu.make_async_copy(
                table_hbm.at[idx_v.at[0]], rows[b], gsem[b]
            ).wait()

        def store(j, b):
            pltpu.make_async_copy(
                rows[b], out_hbm.at[pl.ds((base + j) * CHUNK, CHUNK)], ssem[b]
            ).start()

        def store_wait(b):
            pltpu.make_async_copy(
                rows[b], out_hbm.at[pl.ds(0, CHUNK)], ssem[b]
            ).wait()

        for b in range(NBUF):
            gather(b, b)

        def group(g, carry):
            for b in range(NBUF):
                gather_wait(b)
                store(g * NBUF + b, b)
            for b in range(NBUF):
                store_wait(b)
                gather((g + 1) * NBUF + b, b)
            return carry

        lax.fori_loop(0, GROUPS - 1, group, 0)

        for b in range(NBUF):
            gather_wait(b)
            store((GROUPS - 1) * NBUF + b, b)
        for b in range(NBUF):
            store_wait(b)

    return gather_kernel


_gather = _make_gather()


@jax.jit
def kernel(x, embeddings):
    t4 = jnp.concatenate([embeddings, jnp.roll(embeddings, -1, axis=0)], axis=1)
    idx = x.reshape(N_CHUNKS, CHUNK)
    lines = _gather(t4, idx)
    return lines[:, :DEPTH].reshape(x.shape[0], x.shape[1], DEPTH)


# speedup vs baseline: 1.0621x; 1.0579x over previous
import functools

import jax
import jax.numpy as jnp
from jax import lax
from jax.experimental import pallas as pl
from jax.experimental.pallas import tpu as pltpu
from jax.experimental.pallas import tpu_sc as plsc

VOCAB = 100000
DEPTH = 64
LINE = 2 * DEPTH
B_TOTAL = 4096 * 200
CHUNK = 128
N_CHUNKS = B_TOTAL // CHUNK
NC, NS = 2, 16
NW = NC * NS
CPW = N_CHUNKS // NW
NBUF = 5
GROUPS = CPW // NBUF


def _make_gather():
    mesh = plsc.VectorSubcoreMesh(core_axis_name="c", subcore_axis_name="s")

    @functools.partial(
        pl.kernel,
        mesh=mesh,
        out_type=jax.ShapeDtypeStruct((B_TOTAL, LINE), jnp.float32),
        scratch_types=[
            pltpu.VMEM((CPW, CHUNK), jnp.int32),
            [pltpu.VMEM((CHUNK, LINE), jnp.float32) for _ in range(NBUF)],
            [pltpu.SemaphoreType.DMA for _ in range(NBUF)],
            [pltpu.SemaphoreType.DMA for _ in range(NBUF)],
        ],
        compiler_params=pltpu.CompilerParams(use_tc_tiling_on_sc=True),
    )
    def gather_kernel(table_hbm, idx_hbm, out_hbm, idx_v, rows, gsem, ssem):
        wid = lax.axis_index("s") * NC + lax.axis_index("c")
        base = wid * CPW
        pltpu.sync_copy(idx_hbm.at[pl.ds(base, CPW)], idx_v)

        def gather(j, b):
            pltpu.make_async_copy(
                table_hbm.at[idx_v.at[j]], rows[b], gsem[b]
            ).start()

        def gather_wait(b):
            pltpu.make_async_copy(
                table_hbm.at[idx_v.at[0]], rows[b], gsem[b]
            ).wait()

        def store(j, b):
            pltpu.make_async_copy(
                rows[b], out_hbm.at[pl.ds((base + j) * CHUNK, CHUNK)], ssem[b]
            ).start()

        def store_wait(b):
            pltpu.make_async_copy(
                rows[b], out_hbm.at[pl.ds(0, CHUNK)], ssem[b]
            ).wait()

        for b in range(NBUF):
            gather(b, b)

        def group(g, carry):
            for b in range(NBUF):
                gather_wait(b)
                store(g * NBUF + b, b)
            for b in range(NBUF):
                store_wait(b)
                gather((g + 1) * NBUF + b, b)
            return carry

        lax.fori_loop(0, GROUPS - 1, group, 0)

        for b in range(NBUF):
            gather_wait(b)
            store((GROUPS - 1) * NBUF + b, b)
        for b in range(NBUF):
            store_wait(b)

    return gather_kernel


_gather = _make_gather()


@jax.jit
def kernel(x, embeddings):
    t4 = jnp.concatenate([embeddings, embeddings], axis=1)
    idx = x.reshape(N_CHUNKS, CHUNK)
    lines = _gather(t4, idx)
    return lines[:, :DEPTH].reshape(x.shape[0], x.shape[1], DEPTH)
